# baseline (speedup 1.0000x reference)
import functools

import jax
import jax.numpy as jnp
from jax import lax
from jax.experimental import pallas as pl
from jax.experimental.pallas import tpu as pltpu
from jax.experimental.pallas import tpu_sc as plsc

_NUM_EMBED = 8192
_DIM = 256
_COMMIT = 0.25

_NB = 576
_KB = 2048


def _argmin_body(x_ref, e2_ref, en_ref, idx_ref, loss_ref, loss_acc):
    n = pl.program_id(0)
    nn = pl.num_programs(0)
    xdl = x_ref[0]
    rowsq = jnp.sum(xdl * xdl, axis=0)[:, None]
    cols = lax.broadcasted_iota(jnp.int32, (_NB, _KB), 1).astype(jnp.float32)

    best_val = None
    best_idx = None
    for j in range(_NUM_EMBED // _KB):
        enorm = en_ref[j, :]
        prod2 = lax.dot_general(xdl, e2_ref[pl.ds(j * _KB, _KB), :],
                                (((0,), (1,)), ((), ())),
                                preferred_element_type=jnp.float32)
        scores = (rowsq + enorm[None, :]) - prod2
        bmin = jnp.min(scores, axis=1, keepdims=True)
        bidx = jnp.min(jnp.where(scores == bmin, cols, float(_NUM_EMBED)),
                       axis=1, keepdims=True) + float(j * _KB)
        if j == 0:
            best_val, best_idx = bmin, bidx
        else:
            upd = bmin < best_val
            best_val = jnp.where(upd, bmin, best_val)
            best_idx = jnp.where(upd, bidx, best_idx)

    idx_ref[...] = best_idx.astype(jnp.int32)
    partial = jnp.sum(best_val)
    tot = jnp.where(n == 0, 0.0, loss_acc[0, 0]) + partial
    loss_acc[0, 0] = tot

    @pl.when(n == nn - 1)
    def _():
        loss_ref[0, 0] = tot


def _argmin_call(x, emb, interpret=False):
    b = x.shape[0]
    n_tok = b * x.shape[2]
    e2 = emb + emb
    enorm = jnp.sum(emb ** 2, axis=1).reshape(_NUM_EMBED // _KB, _KB)
    return pl.pallas_call(
        _argmin_body,
        grid=(b,),
        in_specs=[
            pl.BlockSpec((1, _DIM, _NB), lambda n: (n, 0, 0)),
            pl.BlockSpec((_NUM_EMBED, _DIM), lambda n: (0, 0)),
            pl.BlockSpec((_NUM_EMBED // _KB, _KB), lambda n: (0, 0)),
        ],
        out_specs=[
            pl.BlockSpec((_NB, 1), lambda n: (n, 0)),
            pl.BlockSpec((1, 1), lambda n: (0, 0), memory_space=pltpu.SMEM),
        ],
        out_shape=[
            jax.ShapeDtypeStruct((n_tok, 1), jnp.int32),
            jax.ShapeDtypeStruct((1, 1), jnp.float32),
        ],
        scratch_shapes=[
            pltpu.SMEM((1, 1), jnp.float32),
        ],
        interpret=interpret,
    )(x, e2, enorm)


def _make_gather(n_tok):
    info = plsc.get_sparse_core_info()
    nc, ns = info.num_cores, info.num_subcores
    nw = nc * ns
    b_per_w = n_tok // nw
    mesh = plsc.VectorSubcoreMesh(core_axis_name="c", subcore_axis_name="s")

    @functools.partial(
        pl.kernel,
        mesh=mesh,
        out_type=jax.ShapeDtypeStruct((n_tok, _DIM), jnp.float32),
        scratch_types=[
            pltpu.VMEM((b_per_w,), jnp.int32),
            pltpu.VMEM((b_per_w, _DIM), jnp.float32),
            pltpu.SemaphoreType.DMA,
        ],
    )
    def gather(table_hbm, idx_hbm, out_hbm, idx_v, rows_v, sem):
        wid = lax.axis_index("s") * nc + lax.axis_index("c")
        base = wid * b_per_w
        pltpu.sync_copy(idx_hbm.at[pl.ds(base, b_per_w)], idx_v)
        pltpu.async_copy(table_hbm.at[idx_v], rows_v, sem).wait()
        pltpu.sync_copy(rows_v, out_hbm.at[pl.ds(base, b_per_w)])

    return gather


def kernel(x, embedding_weight):
    b, d, l = x.shape
    n_tok = b * l

    idx2d, loss11 = _argmin_call(x, embedding_weight)
    loss = _COMMIT * loss11[0, 0] / (n_tok * _DIM)

    q = _make_gather(n_tok)(embedding_weight, idx2d.reshape(-1))
    quantizer = q.reshape(b, d, l)
    return (quantizer, loss, idx2d)


# speedup vs baseline: 1.0337x; 1.0337x over previous
import functools

import jax
import jax.numpy as jnp
from jax import lax
from jax.experimental import pallas as pl
from jax.experimental.pallas import tpu as pltpu
from jax.experimental.pallas import tpu_sc as plsc

_NUM_EMBED = 8192
_DIM = 256
_COMMIT = 0.25

_NB = 576
_KB = 2048


def _argmin_body(x_ref, e_ref, idx_ref, loss_ref, en_ref, e2_ref, loss_acc):
    n = pl.program_id(0)
    nn = pl.num_programs(0)
    xdl = x_ref[0]
    rowsq = jnp.sum(xdl * xdl, axis=0)[:, None]
    cols = lax.broadcasted_iota(jnp.int32, (_NB, _KB), 1).astype(jnp.float32)

    @pl.when(n == 0)
    def _():
        for j in range(_NUM_EMBED // _KB):
            eb = e_ref[pl.ds(j * _KB, _KB), :]
            en_ref[j, :] = jnp.sum(eb * eb, axis=1)
            e2_ref[pl.ds(j * _KB, _KB), :] = eb + eb

    best_val = None
    best_idx = None
    for j in range(_NUM_EMBED // _KB):
        enorm = en_ref[j, :]
        prod2 = lax.dot_general(xdl, e2_ref[pl.ds(j * _KB, _KB), :],
                                (((0,), (1,)), ((), ())),
                                preferred_element_type=jnp.float32)
        scores = (rowsq + enorm[None, :]) - prod2
        bmin = jnp.min(scores, axis=1, keepdims=True)
        bidx = jnp.min(jnp.where(scores == bmin, cols, float(_NUM_EMBED)),
                       axis=1, keepdims=True) + float(j * _KB)
        if j == 0:
            best_val, best_idx = bmin, bidx
        else:
            upd = bmin < best_val
            best_val = jnp.where(upd, bmin, best_val)
            best_idx = jnp.where(upd, bidx, best_idx)

    idx_ref[...] = best_idx.astype(jnp.int32)
    partial = jnp.sum(best_val)
    tot = jnp.where(n == 0, 0.0, loss_acc[0, 0]) + partial
    loss_acc[0, 0] = tot

    @pl.when(n == nn - 1)
    def _():
        loss_ref[0, 0] = tot


def _argmin_call(x, emb, interpret=False):
    b = x.shape[0]
    n_tok = b * x.shape[2]
    return pl.pallas_call(
        _argmin_body,
        grid=(b,),
        in_specs=[
            pl.BlockSpec((1, _DIM, _NB), lambda n: (n, 0, 0)),
            pl.BlockSpec((_NUM_EMBED, _DIM), lambda n: (0, 0)),
        ],
        out_specs=[
            pl.BlockSpec((_NB, 1), lambda n: (n, 0)),
            pl.BlockSpec((1, 1), lambda n: (0, 0), memory_space=pltpu.SMEM),
        ],
        out_shape=[
            jax.ShapeDtypeStruct((n_tok, 1), jnp.int32),
            jax.ShapeDtypeStruct((1, 1), jnp.float32),
        ],
        scratch_shapes=[
            pltpu.VMEM((_NUM_EMBED // _KB, _KB), jnp.float32),
            pltpu.VMEM((_NUM_EMBED, _DIM), jnp.float32),
            pltpu.SMEM((1, 1), jnp.float32),
        ],
        interpret=interpret,
    )(x, emb)


def _make_gather(n_tok):
    info = plsc.get_sparse_core_info()
    nc, ns = info.num_cores, info.num_subcores
    nw = nc * ns
    b_per_w = n_tok // nw
    mesh = plsc.VectorSubcoreMesh(core_axis_name="c", subcore_axis_name="s")

    @functools.partial(
        pl.kernel,
        mesh=mesh,
        out_type=jax.ShapeDtypeStruct((n_tok, _DIM), jnp.float32),
        scratch_types=[
            pltpu.VMEM((b_per_w,), jnp.int32),
            pltpu.VMEM((b_per_w, _DIM), jnp.float32),
            pltpu.SemaphoreType.DMA,
        ],
    )
    def gather(table_hbm, idx_hbm, out_hbm, idx_v, rows_v, sem):
        wid = lax.axis_index("s") * nc + lax.axis_index("c")
        base = wid * b_per_w
        pltpu.sync_copy(idx_hbm.at[pl.ds(base, b_per_w)], idx_v)
        pltpu.async_copy(table_hbm.at[idx_v], rows_v, sem).wait()
        pltpu.sync_copy(rows_v, out_hbm.at[pl.ds(base, b_per_w)])

    return gather


def kernel(x, embedding_weight):
    b, d, l = x.shape
    n_tok = b * l

    idx2d, loss11 = _argmin_call(x, embedding_weight)
    loss = _COMMIT * loss11[0, 0] / (n_tok * _DIM)

    q = _make_gather(n_tok)(embedding_weight, idx2d.reshape(-1))
    quantizer = q.reshape(b, d, l)
    return (quantizer, loss, idx2d)


# speedup vs baseline: 1.0780x; 1.0429x over previous
import functools

import jax
import jax.numpy as jnp
from jax import lax
from jax.experimental import pallas as pl
from jax.experimental.pallas import tpu as pltpu
from jax.experimental.pallas import tpu_sc as plsc

_NUM_EMBED = 8192
_DIM = 256
_COMMIT = 0.25

_NB = 576
_KB = 1024


def _argmin_body(x_ref, e_ref, en_in_ref, idx_ref, loss_ref, e2_ref, loss_acc):
    n = pl.program_id(0)
    nn = pl.num_programs(0)
    xdl = x_ref[0]
    rowsq = jnp.sum(xdl * xdl, axis=0)[:, None]
    cols = lax.broadcasted_iota(jnp.int32, (_NB, _KB), 1).astype(jnp.float32)

    @pl.when(n == 0)
    def _():
        for j in range(_NUM_EMBED // _KB):
            eb = e_ref[pl.ds(j * _KB, _KB), :]
            e2_ref[pl.ds(j * _KB, _KB), :] = eb + eb

    best_val = None
    best_idx = None
    for j in range(_NUM_EMBED // _KB):
        enorm = en_in_ref[j, :]
        prod2 = lax.dot_general(xdl, e2_ref[pl.ds(j * _KB, _KB), :],
                                (((0,), (1,)), ((), ())),
                                preferred_element_type=jnp.float32)
        scores = (rowsq + enorm[None, :]) - prod2
        bmin = jnp.min(scores, axis=1, keepdims=True)
        bidx = jnp.min(jnp.where(scores == bmin, cols, float(_NUM_EMBED)),
                       axis=1, keepdims=True) + float(j * _KB)
        if j == 0:
            best_val, best_idx = bmin, bidx
        else:
            upd = bmin < best_val
            best_val = jnp.where(upd, bmin, best_val)
            best_idx = jnp.where(upd, bidx, best_idx)

    idx_ref[...] = best_idx.astype(jnp.int32)
    partial = jnp.sum(best_val)
    tot = jnp.where(n == 0, 0.0, loss_acc[0, 0]) + partial
    loss_acc[0, 0] = tot

    @pl.when(n == nn - 1)
    def _():
        loss_ref[0, 0] = tot


def _argmin_call(x, emb, interpret=False):
    b = x.shape[0]
    n_tok = b * x.shape[2]
    return pl.pallas_call(
        _argmin_body,
        grid=(b,),
        in_specs=[
            pl.BlockSpec((1, _DIM, _NB), lambda n: (n, 0, 0)),
            pl.BlockSpec((_NUM_EMBED, _DIM), lambda n: (0, 0)),
            pl.BlockSpec((_NUM_EMBED // _KB, _KB), lambda n: (0, 0)),
        ],
        out_specs=[
            pl.BlockSpec((_NB, 1), lambda n: (n, 0)),
            pl.BlockSpec((1, 1), lambda n: (0, 0), memory_space=pltpu.SMEM),
        ],
        out_shape=[
            jax.ShapeDtypeStruct((n_tok, 1), jnp.int32),
            jax.ShapeDtypeStruct((1, 1), jnp.float32),
        ],
        scratch_shapes=[
            pltpu.VMEM((_NUM_EMBED, _DIM), jnp.float32),
            pltpu.SMEM((1, 1), jnp.float32),
        ],
        interpret=interpret,
    )(x, emb, jnp.sum(emb ** 2, axis=1).reshape(_NUM_EMBED // _KB, _KB))


def _make_gather(n_tok):
    info = plsc.get_sparse_core_info()
    nc, ns = info.num_cores, info.num_subcores
    nw = nc * ns
    b_per_w = n_tok // nw
    mesh = plsc.VectorSubcoreMesh(core_axis_name="c", subcore_axis_name="s")

    @functools.partial(
        pl.kernel,
        mesh=mesh,
        out_type=jax.ShapeDtypeStruct((n_tok, _DIM), jnp.float32),
        scratch_types=[
            pltpu.VMEM((b_per_w,), jnp.int32),
            pltpu.VMEM((b_per_w, _DIM), jnp.float32),
            pltpu.SemaphoreType.DMA,
        ],
    )
    def gather(table_hbm, idx_hbm, out_hbm, idx_v, rows_v, sem):
        wid = lax.axis_index("s") * nc + lax.axis_index("c")
        base = wid * b_per_w
        pltpu.sync_copy(idx_hbm.at[pl.ds(base, b_per_w)], idx_v)
        pltpu.async_copy(table_hbm.at[idx_v], rows_v, sem).wait()
        pltpu.sync_copy(rows_v, out_hbm.at[pl.ds(base, b_per_w)])

    return gather


def kernel(x, embedding_weight):
    b, d, l = x.shape
    n_tok = b * l

    idx2d, loss11 = _argmin_call(x, embedding_weight)
    loss = _COMMIT * loss11[0, 0] / (n_tok * _DIM)

    q = _make_gather(n_tok)(embedding_weight, idx2d.reshape(-1))
    quantizer = q.reshape(b, d, l)
    return (quantizer, loss, idx2d)
